# baseline (speedup 1.0000x reference)
import functools

import jax
import jax.numpy as jnp
from jax import lax
from jax.experimental import pallas as pl
from jax.experimental.pallas import tpu as pltpu
from jax.experimental.pallas import tpu_sc as plsc

N_USERS = 10000
N_ITEMS = 40000
N_NODES = 50000
N_EDGES = 800000
EMB = 64
DF = 128
N_CORES = 2
N_SUB = 16

HCOL = EMB // N_CORES
HROWS = 400
NPAD = HROWS * 128
CHUNK = 128
SB = 3
STAGE = SB * CHUNK
N_STAGES = 132
E_PER_SUB = N_STAGES * STAGE
E_PAD = E_PER_SUB * N_SUB
ZROWS = NPAD // N_SUB
E_PER_W = E_PAD // (N_CORES * N_SUB)
HSTG = 1152
N_HSTG = E_PER_W // HSTG
TBLK = 2048
TGRID = NPAD // TBLK


def _make_sc_scatter():
  mesh = plsc.VectorSubcoreMesh(core_axis_name="c", subcore_axis_name="s",
                                num_cores=N_CORES, num_subcores=N_SUB)

  vmem_i = lambda n: pltpu.VMEM((n,), jnp.int32)

  @functools.partial(
      pl.kernel,
      out_type=jax.ShapeDtypeStruct((N_CORES, NPAD, HCOL), jnp.float32),
      mesh=mesh,
      compiler_params=pltpu.CompilerParams(use_tc_tiling_on_sc=False),
      scratch_types=[
          pltpu.VMEM_SHARED((NPAD, HCOL), jnp.float32),
          [vmem_i(STAGE), vmem_i(STAGE)],
          [vmem_i(STAGE), vmem_i(STAGE)],
          [[vmem_i(CHUNK) for _ in range(SB)] for _ in range(2)],
          [[pltpu.VMEM((CHUNK, HCOL), jnp.float32) for _ in range(SB)]
           for _ in range(2)],
          [pltpu.SemaphoreType.DMA] * 2,
          [pltpu.SemaphoreType.DMA] * 2,
          [pltpu.SemaphoreType.DMA] * 2,
      ],
  )
  def sc_scatter(yl_hbm, yr_hbm, srcp_hbm, dstp_hbm, zero_hbm, out_hbm,
                 acc, srcst, dstst, ldst, ybuf, gsem, ssem, isem):
    c = lax.axis_index("c")
    s = lax.axis_index("s")
    e0 = s * E_PER_SUB

    def idx_issue(st, p):
      off = pl.multiple_of(e0 + st * STAGE, 8)
      pltpu.async_copy(srcp_hbm.at[pl.ds(off, STAGE)], srcst[p], isem[p])
      pltpu.async_copy(dstp_hbm.at[pl.ds(off, STAGE)], dstst[p], isem[p])

    def idx_wait(p):
      pltpu.make_async_copy(srcp_hbm.at[pl.ds(0, STAGE)], srcst[p],
                            isem[p]).wait()
      pltpu.make_async_copy(dstp_hbm.at[pl.ds(0, STAGE)], dstst[p],
                            isem[p]).wait()

    def run_core(y_hbm, out_view):
      pltpu.sync_copy(zero_hbm, acc.at[pl.ds(s * ZROWS, ZROWS)])
      plsc.subcore_barrier()

      def gather_issue(p, b):
        pltpu.async_copy(y_hbm.at[srcst[p].at[pl.ds(b * CHUNK, CHUNK)]],
                         ybuf[p][b], gsem[p])

      def gather_wait(p, b):
        pltpu.make_async_copy(y_hbm.at[srcst[p].at[pl.ds(b * CHUNK, CHUNK)]],
                              ybuf[p][b], gsem[p]).wait()

      def scatter_issue(p, b):
        pltpu.async_copy(ybuf[p][b], acc.at[ldst[p][b]], ssem[p], add=True)

      def scatter_wait(p, b):
        pltpu.make_async_copy(ybuf[p][b], acc.at[ldst[p][b]], ssem[p]).wait()

      idx_issue(0, 0)
      idx_issue(1, 1)

      def stage_body(s2, carry):
        for p in range(2):
          st = 2 * s2 + p

          @pl.when(s2 >= 1)
          def _():
            for b in range(SB):
              scatter_wait(p, b)

          idx_wait(p)
          for b in range(SB):
            gather_issue(p, b)
          for b in range(SB):
            for k in range(CHUNK // 16):
              ldst[p][b][pl.ds(k * 16, 16)] = (
                  dstst[p][pl.ds(b * CHUNK + k * 16, 16)])
          for b in range(SB):
            gather_wait(p, b)
            scatter_issue(p, b)

          @pl.when(s2 <= (N_STAGES // 2) - 2)
          def _():
            idx_issue(st + 2, p)
        return carry

      lax.fori_loop(0, N_STAGES // 2, stage_body, 0)
      for p in range(2):
        for b in range(SB):
          scatter_wait(p, b)
      plsc.subcore_barrier()

      r0 = s * ZROWS
      pltpu.sync_copy(acc.at[pl.ds(r0, ZROWS)], out_view.at[pl.ds(r0, ZROWS)])
      plsc.subcore_barrier()

    @pl.when(c == 0)
    def _():
      run_core(yl_hbm, out_hbm.at[0])

    @pl.when(c == 1)
    def _():
      run_core(yr_hbm, out_hbm.at[1])

  return sc_scatter


_sc_scatter = _make_sc_scatter()


def _make_sc_hist():
  mesh = plsc.VectorSubcoreMesh(core_axis_name="c", subcore_axis_name="s",
                                num_cores=N_CORES, num_subcores=N_SUB)

  vmem_i = lambda n: pltpu.VMEM((n,), jnp.int32)

  @functools.partial(
      pl.kernel,
      out_type=jax.ShapeDtypeStruct((2, N_CORES, HROWS, 128), jnp.float32),
      mesh=mesh,
      compiler_params=pltpu.CompilerParams(use_tc_tiling_on_sc=False,
                                           needs_layout_passes=False),
      scratch_types=[
          [pltpu.VMEM_SHARED((HROWS, 128), jnp.float32) for _ in range(2)],
          [pltpu.VMEM((HROWS, 128), jnp.float32) for _ in range(2)],
          [vmem_i(HSTG), vmem_i(HSTG)],
          [vmem_i(128), vmem_i(128), vmem_i(128), vmem_i(16)],
      ],
  )
  def sc_hist(srcp_hbm, dstp_hbm, zeroh_hbm, out_hbm, hsh, hloc, stg, ridx):
    c = lax.axis_index("c")
    s = lax.axis_index("s")
    w = c * N_SUB + s
    e0 = w * E_PER_W

    pltpu.sync_copy(zeroh_hbm, hloc[0])
    pltpu.sync_copy(zeroh_hbm, hloc[1])

    @pl.when(s == 0)
    def _():
      pltpu.sync_copy(zeroh_hbm, hsh[0])

    @pl.when(s == 1)
    def _():
      pltpu.sync_copy(zeroh_hbm, hsh[1])

    for t in range(3):
      for j in range(8):
        ridx[t][pl.ds(j * 16, 16)] = (lax.iota(jnp.int32, 16)
                                      + (t * 128 + j * 16))
    ridx[3][pl.ds(0, 16)] = lax.iota(jnp.int32, 16) + 384

    ones16 = jnp.full((16,), 1.0, jnp.float32)

    def stage_body(t, carry):
      off = pl.multiple_of(e0 + t * HSTG, 8)
      pltpu.sync_copy(srcp_hbm.at[pl.ds(off, HSTG)], stg[0])
      pltpu.sync_copy(dstp_hbm.at[pl.ds(off, HSTG)], stg[1])
      for k in range(HSTG // 16):
        sv = stg[0][pl.ds(k * 16, 16)]
        dv = stg[1][pl.ds(k * 16, 16)]
        plsc.addupdate_scatter(
            hloc[0], [lax.shift_right_logical(dv, 7), dv & 127], ones16)
        plsc.addupdate_scatter(
            hloc[1], [lax.shift_right_logical(sv, 7), sv & 127], ones16)
      return carry

    lax.fori_loop(0, N_HSTG, stage_body, 0)
    plsc.subcore_barrier()

    for h in range(2):
      for t in range(3):
        pltpu.sync_copy(hloc[h].at[pl.ds(t * 128, 128)],
                        hsh[h].at[ridx[t]], add=True)
      pltpu.sync_copy(hloc[h].at[pl.ds(384, 16)], hsh[h].at[ridx[3]],
                      add=True)
    plsc.subcore_barrier()

    for cc in range(N_CORES):
      @pl.when((c == cc) & (s == 0))
      def _(cc=cc):
        pltpu.sync_copy(hsh[0], out_hbm.at[0, cc])

      @pl.when((c == cc) & (s == 1))
      def _(cc=cc):
        pltpu.sync_copy(hsh[1], out_hbm.at[1, cc])

  return sc_hist


_sc_hist = _make_sc_hist()


def _proj_body(f_ref, w_ref, o_ref):
  o_ref[...] = jnp.dot(f_ref[...], w_ref[...],
                       preferred_element_type=jnp.float32)


def _proj(feat, w):
  blk = 400
  grid = N_ITEMS // blk
  return pl.pallas_call(
      _proj_body,
      grid=(grid,),
      in_specs=[pl.BlockSpec((blk, DF), lambda i: (i, 0)),
                pl.BlockSpec((DF, EMB), lambda i: (0, 0))],
      out_specs=pl.BlockSpec((blk, EMB), lambda i: (i, 0)),
      out_shape=jax.ShapeDtypeStruct((N_ITEMS, EMB), jnp.float32),
  )(feat, w)


def _degprep_body(p_ref, o_ref):
  o_ref[...] = lax.rsqrt(jnp.sum(p_ref[...], axis=1) + 1.0)


def _degprep(parts):
  return pl.pallas_call(
      _degprep_body,
      grid=(TGRID,),
      in_specs=[pl.BlockSpec((2, N_CORES, TBLK // 128, 128),
                             lambda i: (0, 0, i, 0))],
      out_specs=pl.BlockSpec((2, TBLK // 128, 128), lambda i: (0, i, 0)),
      out_shape=jax.ShapeDtypeStruct((2, HROWS, 128), jnp.float32),
  )(parts)


def _deg_spec():
  return pl.BlockSpec((TBLK, 1), lambda i: (i, 0))


def _scale_body(x_ref, d_ref, yl_ref, yr_ref):
  y = x_ref[...] * d_ref[...]
  yl_ref[...] = y[:, :HCOL]
  yr_ref[...] = y[:, HCOL:]


def _scale(x, dout2):
  half = pl.BlockSpec((TBLK, HCOL), lambda i: (i, 0))
  return pl.pallas_call(
      _scale_body,
      grid=(TGRID,),
      in_specs=[pl.BlockSpec((TBLK, EMB), lambda i: (i, 0)), _deg_spec()],
      out_specs=[half, half],
      out_shape=[jax.ShapeDtypeStruct((NPAD, HCOL), jnp.float32),
                 jax.ShapeDtypeStruct((NPAD, HCOL), jnp.float32)],
  )(x, dout2)


def _make_combine_body(want_x, want_y):
  def body(agg2_ref, yl_ref, yr_ref, ne_ref, din_ref, dout_ref,
           wg_ref, wi_ref, b_ref, *out_refs):
    a2 = agg2_ref[...]
    agg0 = jnp.concatenate([a2[0], a2[1]], axis=-1)
    y = jnp.concatenate([yl_ref[...], yr_ref[...]], axis=-1)
    agg = (agg0 + y) * din_ref[...]
    z = (jnp.dot(agg, wg_ref[...], preferred_element_type=jnp.float32)
         + jnp.dot(ne_ref[...], wi_ref[...],
                   preferred_element_type=jnp.float32)
         + b_ref[...])
    x = jnp.where(z >= 0, z, 0.01 * z)
    outs = list(out_refs)
    if want_x:
      outs.pop(0)[...] = x
    if want_y:
      yn = x * dout_ref[...]
      outs[0][...] = yn[:, :HCOL]
      outs[1][...] = yn[:, HCOL:]
  return body


def _combine(agg2, y_l, y_r, node_emb, din2, dout2, wg, wi, b,
             want_x=True, want_y=True):
  half = pl.BlockSpec((TBLK, HCOL), lambda i: (i, 0))
  pair = pl.BlockSpec((N_CORES, TBLK, HCOL), lambda i: (0, i, 0))
  full = pl.BlockSpec((TBLK, EMB), lambda i: (i, 0))
  out_specs = ([full] if want_x else []) + ([half, half] if want_y else [])
  out_shape = (([jax.ShapeDtypeStruct((NPAD, EMB), jnp.float32)]
                if want_x else [])
               + ([jax.ShapeDtypeStruct((NPAD, HCOL), jnp.float32)] * 2
                  if want_y else []))
  return pl.pallas_call(
      _make_combine_body(want_x, want_y),
      grid=(TGRID,),
      in_specs=[pair, half, half, full,
                _deg_spec(), _deg_spec(),
                pl.BlockSpec((EMB, EMB), lambda i: (0, 0)),
                pl.BlockSpec((EMB, EMB), lambda i: (0, 0)),
                pl.BlockSpec((1, EMB), lambda i: (0, 0))],
      out_specs=out_specs,
      out_shape=out_shape,
  )(agg2, y_l, y_r, node_emb, din2, dout2, wg, wi, b.reshape(1, EMB))


def _fuse_body(a_ref, b_ref, emb_ref, pres_ref):
  m = pl.program_id(0)
  a = a_ref[...]
  b = b_ref[...]
  emb_ref[...] = 0.5 * (a + b)
  pres_ref[...] = jnp.where(m == 0, a, b)[None]


def _fuse_stack(a, b):
  blk = 400
  row = pl.BlockSpec((blk, EMB), lambda m, i: (i, 0))
  return pl.pallas_call(
      _fuse_body,
      grid=(2, N_NODES // blk),
      in_specs=[row, row],
      out_specs=[pl.BlockSpec((blk, EMB), lambda m, i: (i, 0)),
                 pl.BlockSpec((1, blk, EMB), lambda m, i: (m, i, 0))],
      out_shape=[jax.ShapeDtypeStruct((N_NODES, EMB), jnp.float32),
                 jax.ShapeDtypeStruct((2, N_NODES, EMB), jnp.float32)],
  )(a, b)


@jax.jit
def kernel(feat_0, feat_1, node_emb, edge_index, user_pref, W_proj,
           W_gcn, W_id, b):
  src = edge_index[0]
  dst = edge_index[1]
  pad_n = E_PAD - N_EDGES
  pad_idx = jnp.full((pad_n,), N_NODES, jnp.int32)
  srcp = jnp.concatenate([src, pad_idx])
  dstp = jnp.concatenate([dst, pad_idx])

  zero_rows = jnp.zeros((ZROWS, HCOL), jnp.float32)
  node_pad = jnp.zeros((NPAD - N_NODES, EMB), jnp.float32)
  node_emb_p = jnp.concatenate([node_emb, node_pad], axis=0)

  hist = _sc_hist(srcp, dstp, jnp.zeros((HROWS, 128), jnp.float32))
  dinv = _degprep(hist).reshape(2, NPAD, 1)
  din2 = dinv[0]
  dout2 = dinv[1]

  feats = (feat_0, feat_1)
  xs = []
  ys = []
  for m in range(2):
    item_h = _proj(feats[m], W_proj[m])
    x = jnp.concatenate([user_pref[m], item_h, node_pad], axis=0)
    xs.append(x)
    ys.append(_scale(x, dout2))
  for l in range(2):
    aggs = [_sc_scatter(ys[m][0], ys[m][1], srcp, dstp, zero_rows)
            for m in range(2)]
    last = l == 1
    for m in range(2):
      outs = _combine(aggs[m], ys[m][0], ys[m][1], node_emb_p,
                      din2, dout2, W_gcn[m, l], W_id[m, l], b[m, l],
                      want_x=last, want_y=not last)
      if last:
        xs[m] = outs[0]
      else:
        ys[m] = (outs[0], outs[1])
  emb, pres = _fuse_stack(xs[0], xs[1])
  return emb, pres


# speedup vs baseline: 1.0646x; 1.0646x over previous
import functools

import jax
import jax.numpy as jnp
from jax import lax
from jax.experimental import pallas as pl
from jax.experimental.pallas import tpu as pltpu
from jax.experimental.pallas import tpu_sc as plsc

N_USERS = 10000
N_ITEMS = 40000
N_NODES = 50000
N_EDGES = 800000
EMB = 64
DF = 128
N_CORES = 2
N_SUB = 16

HCOL = EMB // N_CORES
HROWS = 400
NPAD = HROWS * 128
CHUNK = 128
SB = 3
STAGE = SB * CHUNK
N_STAGES = 132
E_PER_SUB = N_STAGES * STAGE
E_PAD = E_PER_SUB * N_SUB
ZROWS = NPAD // N_SUB
E_PER_W = E_PAD // (N_CORES * N_SUB)
HSTG = 1152
N_HSTG = E_PER_W // HSTG
TBLK = 2048
TGRID = NPAD // TBLK


def _make_sc_scatter():
  mesh = plsc.VectorSubcoreMesh(core_axis_name="c", subcore_axis_name="s",
                                num_cores=N_CORES, num_subcores=N_SUB)

  vmem_i = lambda n: pltpu.VMEM((n,), jnp.int32)

  @functools.partial(
      pl.kernel,
      out_type=jax.ShapeDtypeStruct((N_CORES, NPAD, HCOL), jnp.float32),
      mesh=mesh,
      compiler_params=pltpu.CompilerParams(use_tc_tiling_on_sc=False),
      scratch_types=[
          pltpu.VMEM_SHARED((NPAD, HCOL), jnp.float32),
          [vmem_i(STAGE), vmem_i(STAGE)],
          [vmem_i(STAGE), vmem_i(STAGE)],
          [[vmem_i(CHUNK) for _ in range(SB)] for _ in range(2)],
          [[pltpu.VMEM((CHUNK, HCOL), jnp.float32) for _ in range(SB)]
           for _ in range(2)],
          [pltpu.SemaphoreType.DMA] * 2,
          [pltpu.SemaphoreType.DMA] * 2,
          [pltpu.SemaphoreType.DMA] * 2,
      ],
  )
  def sc_scatter(yl_hbm, yr_hbm, srcp_hbm, dstp_hbm, zero_hbm, out_hbm,
                 acc, srcst, dstst, ldst, ybuf, gsem, ssem, isem):
    c = lax.axis_index("c")
    s = lax.axis_index("s")
    e0 = s * E_PER_SUB

    def idx_issue(st, p):
      off = pl.multiple_of(e0 + st * STAGE, 8)
      pltpu.async_copy(srcp_hbm.at[pl.ds(off, STAGE)], srcst[p], isem[p])
      pltpu.async_copy(dstp_hbm.at[pl.ds(off, STAGE)], dstst[p], isem[p])

    def idx_wait(p):
      pltpu.make_async_copy(srcp_hbm.at[pl.ds(0, STAGE)], srcst[p],
                            isem[p]).wait()
      pltpu.make_async_copy(dstp_hbm.at[pl.ds(0, STAGE)], dstst[p],
                            isem[p]).wait()

    def run_core(y_hbm, out_view):
      pltpu.sync_copy(zero_hbm, acc.at[pl.ds(s * ZROWS, ZROWS)])
      plsc.subcore_barrier()

      def gather_issue(p, b):
        pltpu.async_copy(y_hbm.at[srcst[p].at[pl.ds(b * CHUNK, CHUNK)]],
                         ybuf[p][b], gsem[p])

      def gather_wait(p, b):
        pltpu.make_async_copy(y_hbm.at[srcst[p].at[pl.ds(b * CHUNK, CHUNK)]],
                              ybuf[p][b], gsem[p]).wait()

      def scatter_issue(p, b):
        pltpu.async_copy(ybuf[p][b], acc.at[ldst[p][b]], ssem[p], add=True)

      def scatter_wait(p, b):
        pltpu.make_async_copy(ybuf[p][b], acc.at[ldst[p][b]], ssem[p]).wait()

      idx_issue(0, 0)
      idx_issue(1, 1)

      def stage_body(s2, carry):
        for p in range(2):
          st = 2 * s2 + p

          @pl.when(s2 >= 1)
          def _():
            for b in range(SB):
              scatter_wait(p, b)

          idx_wait(p)
          for b in range(SB):
            gather_issue(p, b)
          for b in range(SB):
            for k in range(CHUNK // 16):
              ldst[p][b][pl.ds(k * 16, 16)] = (
                  dstst[p][pl.ds(b * CHUNK + k * 16, 16)])
          for b in range(SB):
            gather_wait(p, b)
            scatter_issue(p, b)

          @pl.when(s2 <= (N_STAGES // 2) - 2)
          def _():
            idx_issue(st + 2, p)
        return carry

      lax.fori_loop(0, N_STAGES // 2, stage_body, 0)
      for p in range(2):
        for b in range(SB):
          scatter_wait(p, b)
      plsc.subcore_barrier()

      r0 = s * ZROWS
      pltpu.sync_copy(acc.at[pl.ds(r0, ZROWS)], out_view.at[pl.ds(r0, ZROWS)])
      plsc.subcore_barrier()

    @pl.when(c == 0)
    def _():
      run_core(yl_hbm, out_hbm.at[0])

    @pl.when(c == 1)
    def _():
      run_core(yr_hbm, out_hbm.at[1])

  return sc_scatter


_sc_scatter = _make_sc_scatter()


def _make_sc_hist():
  mesh = plsc.VectorSubcoreMesh(core_axis_name="c", subcore_axis_name="s",
                                num_cores=N_CORES, num_subcores=N_SUB)

  vmem_i = lambda n: pltpu.VMEM((n,), jnp.int32)

  @functools.partial(
      pl.kernel,
      out_type=jax.ShapeDtypeStruct((2, N_CORES, HROWS, 128), jnp.float32),
      mesh=mesh,
      compiler_params=pltpu.CompilerParams(use_tc_tiling_on_sc=False,
                                           needs_layout_passes=False),
      scratch_types=[
          [pltpu.VMEM_SHARED((HROWS, 128), jnp.float32) for _ in range(2)],
          [pltpu.VMEM((HROWS, 128), jnp.float32) for _ in range(2)],
          [vmem_i(HSTG), vmem_i(HSTG)],
          [vmem_i(128), vmem_i(128), vmem_i(128), vmem_i(16)],
      ],
  )
  def sc_hist(srcp_hbm, dstp_hbm, zeroh_hbm, out_hbm, hsh, hloc, stg, ridx):
    c = lax.axis_index("c")
    s = lax.axis_index("s")
    w = c * N_SUB + s
    e0 = w * E_PER_W

    pltpu.sync_copy(zeroh_hbm, hloc[0])
    pltpu.sync_copy(zeroh_hbm, hloc[1])

    @pl.when(s == 0)
    def _():
      pltpu.sync_copy(zeroh_hbm, hsh[0])

    @pl.when(s == 1)
    def _():
      pltpu.sync_copy(zeroh_hbm, hsh[1])

    for t in range(3):
      for j in range(8):
        ridx[t][pl.ds(j * 16, 16)] = (lax.iota(jnp.int32, 16)
                                      + (t * 128 + j * 16))
    ridx[3][pl.ds(0, 16)] = lax.iota(jnp.int32, 16) + 384

    ones16 = jnp.full((16,), 1.0, jnp.float32)

    def stage_body(t, carry):
      off = pl.multiple_of(e0 + t * HSTG, 8)
      pltpu.sync_copy(srcp_hbm.at[pl.ds(off, HSTG)], stg[0])
      pltpu.sync_copy(dstp_hbm.at[pl.ds(off, HSTG)], stg[1])
      for k in range(HSTG // 16):
        sv = stg[0][pl.ds(k * 16, 16)]
        dv = stg[1][pl.ds(k * 16, 16)]
        plsc.addupdate_scatter(
            hloc[0], [lax.shift_right_logical(dv, 7), dv & 127], ones16)
        plsc.addupdate_scatter(
            hloc[1], [lax.shift_right_logical(sv, 7), sv & 127], ones16)
      return carry

    lax.fori_loop(0, N_HSTG, stage_body, 0)
    plsc.subcore_barrier()

    for h in range(2):
      for t in range(3):
        pltpu.sync_copy(hloc[h].at[pl.ds(t * 128, 128)],
                        hsh[h].at[ridx[t]], add=True)
      pltpu.sync_copy(hloc[h].at[pl.ds(384, 16)], hsh[h].at[ridx[3]],
                      add=True)
    plsc.subcore_barrier()

    for cc in range(N_CORES):
      @pl.when((c == cc) & (s == 0))
      def _(cc=cc):
        pltpu.sync_copy(hsh[0], out_hbm.at[0, cc])

      @pl.when((c == cc) & (s == 1))
      def _(cc=cc):
        pltpu.sync_copy(hsh[1], out_hbm.at[1, cc])

  return sc_hist


_sc_hist = _make_sc_hist()


def _proj_body(f_ref, w_ref, o_ref):
  o_ref[...] = jnp.dot(f_ref[...], w_ref[...],
                       preferred_element_type=jnp.float32)


def _proj(feat, w):
  blk = 400
  grid = N_ITEMS // blk
  return pl.pallas_call(
      _proj_body,
      grid=(grid,),
      in_specs=[pl.BlockSpec((blk, DF), lambda i: (i, 0)),
                pl.BlockSpec((DF, EMB), lambda i: (0, 0))],
      out_specs=pl.BlockSpec((blk, EMB), lambda i: (i, 0)),
      out_shape=jax.ShapeDtypeStruct((N_ITEMS, EMB), jnp.float32),
  )(feat, w)


def _degprep_body(p_ref, o_ref):
  o_ref[...] = lax.rsqrt(jnp.sum(p_ref[...], axis=1) + 1.0)


def _degprep(parts):
  return pl.pallas_call(
      _degprep_body,
      grid=(TGRID,),
      in_specs=[pl.BlockSpec((2, N_CORES, TBLK // 128, 128),
                             lambda i: (0, 0, i, 0))],
      out_specs=pl.BlockSpec((2, TBLK // 128, 128), lambda i: (0, i, 0)),
      out_shape=jax.ShapeDtypeStruct((2, HROWS, 128), jnp.float32),
  )(parts)


def _deg_spec():
  return pl.BlockSpec((TBLK, 1), lambda i: (i, 0))


def _scale_body(x_ref, d_ref, yl_ref, yr_ref):
  y = x_ref[...] * d_ref[...]
  yl_ref[...] = y[:, :HCOL]
  yr_ref[...] = y[:, HCOL:]


def _scale(x, dout2):
  half = pl.BlockSpec((TBLK, HCOL), lambda i: (i, 0))
  return pl.pallas_call(
      _scale_body,
      grid=(TGRID,),
      in_specs=[pl.BlockSpec((TBLK, EMB), lambda i: (i, 0)), _deg_spec()],
      out_specs=[half, half],
      out_shape=[jax.ShapeDtypeStruct((NPAD, HCOL), jnp.float32),
                 jax.ShapeDtypeStruct((NPAD, HCOL), jnp.float32)],
  )(x, dout2)


def _make_combine_body(want_x, want_y):
  def body(agg2_ref, yl_ref, yr_ref, ne_ref, din_ref, dout_ref,
           wg_ref, wi_ref, b_ref, *out_refs):
    a2 = agg2_ref[...]
    agg0 = jnp.concatenate([a2[0], a2[1]], axis=-1)
    y = jnp.concatenate([yl_ref[...], yr_ref[...]], axis=-1)
    agg = (agg0 + y) * din_ref[...]
    z = (jnp.dot(agg, wg_ref[...], preferred_element_type=jnp.float32)
         + jnp.dot(ne_ref[...], wi_ref[...],
                   preferred_element_type=jnp.float32)
         + b_ref[...])
    x = jnp.where(z >= 0, z, 0.01 * z)
    outs = list(out_refs)
    if want_x:
      outs.pop(0)[...] = x
    if want_y:
      yn = x * dout_ref[...]
      outs[0][...] = yn[:, :HCOL]
      outs[1][...] = yn[:, HCOL:]
  return body


def _combine(agg2, y_l, y_r, node_emb, din2, dout2, wg, wi, b,
             want_x=True, want_y=True):
  half = pl.BlockSpec((TBLK, HCOL), lambda i: (i, 0))
  pair = pl.BlockSpec((N_CORES, TBLK, HCOL), lambda i: (0, i, 0))
  full = pl.BlockSpec((TBLK, EMB), lambda i: (i, 0))
  out_specs = ([full] if want_x else []) + ([half, half] if want_y else [])
  out_shape = (([jax.ShapeDtypeStruct((NPAD, EMB), jnp.float32)]
                if want_x else [])
               + ([jax.ShapeDtypeStruct((NPAD, HCOL), jnp.float32)] * 2
                  if want_y else []))
  return pl.pallas_call(
      _make_combine_body(want_x, want_y),
      grid=(TGRID,),
      in_specs=[pair, half, half, full,
                _deg_spec(), _deg_spec(),
                pl.BlockSpec((EMB, EMB), lambda i: (0, 0)),
                pl.BlockSpec((EMB, EMB), lambda i: (0, 0)),
                pl.BlockSpec((1, EMB), lambda i: (0, 0))],
      out_specs=out_specs,
      out_shape=out_shape,
  )(agg2, y_l, y_r, node_emb, din2, dout2, wg, wi, b.reshape(1, EMB))


def _fuse_body(a_ref, b_ref, o_ref):
  o_ref[...] = 0.5 * (a_ref[...] + b_ref[...])


def _fuse(a, b):
  row = pl.BlockSpec((TBLK, EMB), lambda i: (i, 0))
  return pl.pallas_call(
      _fuse_body,
      grid=(TGRID,),
      in_specs=[row, row],
      out_specs=row,
      out_shape=jax.ShapeDtypeStruct((NPAD, EMB), jnp.float32),
  )(a, b)


@jax.jit
def kernel(feat_0, feat_1, node_emb, edge_index, user_pref, W_proj,
           W_gcn, W_id, b):
  src = edge_index[0]
  dst = edge_index[1]
  pad_n = E_PAD - N_EDGES
  pad_idx = jnp.full((pad_n,), N_NODES, jnp.int32)
  srcp = jnp.concatenate([src, pad_idx])
  dstp = jnp.concatenate([dst, pad_idx])

  zero_rows = jnp.zeros((ZROWS, HCOL), jnp.float32)
  node_pad = jnp.zeros((NPAD - N_NODES, EMB), jnp.float32)
  node_emb_p = jnp.concatenate([node_emb, node_pad], axis=0)

  hist = _sc_hist(srcp, dstp, jnp.zeros((HROWS, 128), jnp.float32))
  dinv = _degprep(hist).reshape(2, NPAD, 1)
  din2 = dinv[0]
  dout2 = dinv[1]

  feats = (feat_0, feat_1)
  xs = []
  ys = []
  for m in range(2):
    item_h = _proj(feats[m], W_proj[m])
    x = jnp.concatenate([user_pref[m], item_h, node_pad], axis=0)
    xs.append(x)
    ys.append(_scale(x, dout2))
  for l in range(2):
    aggs = [_sc_scatter(ys[m][0], ys[m][1], srcp, dstp, zero_rows)
            for m in range(2)]
    last = l == 1
    for m in range(2):
      outs = _combine(aggs[m], ys[m][0], ys[m][1], node_emb_p,
                      din2, dout2, W_gcn[m, l], W_id[m, l], b[m, l],
                      want_x=last, want_y=not last)
      if last:
        xs[m] = outs[0]
      else:
        ys[m] = (outs[0], outs[1])
  emb = _fuse(xs[0], xs[1])[:N_NODES]
  return emb, jnp.stack([xs[0][:N_NODES], xs[1][:N_NODES]])
